# baseline (speedup 1.0000x reference)
import jax
import jax.numpy as jnp
from jax import lax
from jax.experimental import pallas as pl
from jax.experimental.pallas import tpu as pltpu
from jax.experimental.pallas import tpu_sc as plsc

BATCH = 16384
D = 64
WIDE = 2 * D
NC = 2
NS = 16
NW = NC * NS
NROWS = 100000
CW = 128
NCHUNK = NROWS // CW
TAIL0 = NCHUNK * CW
BASE_CH = NCHUNK // NW
EXTRA = NCHUNK - BASE_CH * NW
NVEC = BATCH // 16


def _sc_body(user_h, item_h, guT, giT, muT, miT, tail_u, tail_i,
             u_out, i_out,
             sidx, mlist_u, mlist_i, clist, cbuf_g, cbuf_m,
             stag, posref, rrbuf, sem0, sem1, sem2):
    w = lax.axis_index("s") * NC + lax.axis_index("c")
    lo_chunk = BASE_CH * w + jnp.minimum(w, EXTRA)
    ncz = BASE_CH + (w < EXTRA).astype(jnp.int32)
    lo_col = lo_chunk * CW
    is_last = w == NW - 1
    hi_col = jnp.where(is_last, NROWS, (lo_chunk + ncz) * CW)
    iota = lax.iota(jnp.int32, 16)

    def scan(idx_h, mlist):
        pltpu.sync_copy(idx_h, sidx)

        def body(v, cnt):
            lanes = v * 16 + iota
            r = plsc.load_gather(sidx, [lanes])
            m = (r >= lo_col) & (r < hi_col)
            e = (r << 14) | lanes
            plsc.store_compressed(mlist.at[pl.ds(cnt, 16)], e, mask=m)
            return cnt + jnp.sum(m.astype(jnp.int32))

        return lax.fori_loop(0, NVEC, body, 0)

    n_u = scan(user_h, mlist_u)
    n_i = scan(item_h, mlist_i)

    def subfilter(mlist, n, c):
        def body(v, cnt):
            lanes = v * 16 + iota
            e = plsc.load_gather(mlist, [jnp.minimum(lanes, n - 1)])
            m = (lanes < n) & ((e >> 21) == c)
            plsc.store_compressed(clist.at[pl.ds(cnt, 16)], e, mask=m)
            return cnt + jnp.sum(m.astype(jnp.int32))

        return lax.fori_loop(0, (n + 15) // 16, body, 0)

    def process_side(c, mlist, n, tbl_g, tbl_m, out):
        col0 = pl.multiple_of(c * CW, CW)
        pltpu.sync_copy(tbl_g.at[:, pl.ds(col0, CW)], cbuf_g)
        pltpu.sync_copy(tbl_m.at[:, pl.ds(col0, CW)], cbuf_m)
        g_cnt = subfilter(mlist, n, c)

        def sub_batch(sb, _):
            base = sb * 128
            rrs = []
            for v in range(8):
                lanes = base + v * 16 + iota
                e = plsc.load_gather(clist, [jnp.minimum(lanes, g_cnt - 1)])
                rrs.append((e >> 14) & (CW - 1))
                posref[pl.ds(v * 16, 16)] = e & (BATCH - 1)

            def dloop(d, _2):
                dv = jnp.full((16,), d, jnp.int32)
                for v in range(8):
                    rows = v * 16 + iota
                    g = plsc.load_gather(cbuf_g, [dv, rrs[v]])
                    plsc.store_scatter(stag, [rows, dv], g)
                    m = plsc.load_gather(cbuf_m, [dv, rrs[v]])
                    plsc.store_scatter(stag, [rows, dv + D], m)
                return 0

            lax.fori_loop(0, D, dloop, 0)
            pltpu.async_copy(stag, out.at[posref], sem2).wait()
            return 0

        lax.fori_loop(0, (g_cnt + 127) // 128, sub_batch, 0)

    def chunk_loop(c_rel, _):
        c = lo_chunk + c_rel
        process_side(c, mlist_u, n_u, guT, muT, u_out)
        process_side(c, mlist_i, n_i, giT, miT, i_out)
        return 0

    lax.fori_loop(0, ncz, chunk_loop, 0)

    @pl.when(is_last)
    def _tail():
        def tail_side(mlist, n, tail_t, out):
            g_cnt = subfilter(mlist, n, NCHUNK)

            def sub_batch(sb, _):
                base = sb * 128
                for v in range(8):
                    lanes = base + v * 16 + iota
                    e = plsc.load_gather(clist, [jnp.minimum(lanes, g_cnt - 1)])
                    rrbuf[pl.ds(v * 16, 16)] = (e >> 14) - TAIL0
                    posref[pl.ds(v * 16, 16)] = e & (BATCH - 1)
                pltpu.async_copy(tail_t.at[rrbuf], stag, sem2).wait()
                pltpu.async_copy(stag, out.at[posref], sem2).wait()
                return 0

            lax.fori_loop(0, (g_cnt + 127) // 128, sub_batch, 0)

        tail_side(mlist_u, n_u, tail_u, u_out)
        tail_side(mlist_i, n_i, tail_i, i_out)


def _make_sc():
    mesh = plsc.VectorSubcoreMesh(core_axis_name="c", subcore_axis_name="s")
    f32, i32 = jnp.float32, jnp.int32
    row = jax.ShapeDtypeStruct((BATCH, WIDE), f32)
    return pl.kernel(
        _sc_body,
        out_type=[row, row],
        mesh=mesh,
        scratch_types=[
            pltpu.VMEM((BATCH,), i32),
            pltpu.VMEM((BATCH,), i32),
            pltpu.VMEM((BATCH,), i32),
            pltpu.VMEM((BATCH,), i32),
            pltpu.VMEM((D, CW), f32),
            pltpu.VMEM((D, CW), f32),
            pltpu.VMEM((128, WIDE), f32),
            pltpu.VMEM((128,), i32),
            pltpu.VMEM((128,), i32),
            pltpu.SemaphoreType.DMA,
            pltpu.SemaphoreType.DMA,
            pltpu.SemaphoreType.DMA,
        ],
        compiler_params=pltpu.CompilerParams(use_tc_tiling_on_sc=True,
                                             needs_layout_passes=False),
    )


BLK = 512


def _tc_body(u, i, w0, b0, w1, b1, hw, hb, nw, nb, fused_o, score_o):
    gu = u[:, :D]
    mu = u[:, D:]
    gi = i[:, :D]
    mi = i[:, D:]
    mlp_x = jnp.concatenate([mu, mi], axis=1)
    h = jnp.maximum(jnp.dot(mlp_x, w0[...],
                            preferred_element_type=jnp.float32) + b0[...], 0.0)
    mlp_out = jnp.maximum(jnp.dot(h, w1[...],
                                  preferred_element_type=jnp.float32) + b1[...], 0.0)
    fused_in = jnp.concatenate([0.5 * gu * gi, 0.5 * mlp_out], axis=1)
    fused = jnp.dot(fused_in, hw[...],
                    preferred_element_type=jnp.float32) + hb[...]
    fused_o[...] = fused
    score_o[...] = jnp.sum(fused * nw[...], axis=1) + nb[0, 0]


def _make_tc():
    grid = (BATCH // BLK,)
    blk_in = pl.BlockSpec((BLK, WIDE), lambda i: (i, 0))
    full = lambda shape: pl.BlockSpec(shape, lambda i: (0, 0))
    return pl.pallas_call(
        _tc_body,
        grid=grid,
        in_specs=[
            blk_in, blk_in,
            full((WIDE, WIDE)),
            full((1, WIDE)),
            full((WIDE, D)),
            full((1, D)),
            full((WIDE, D)),
            full((1, D)),
            full((1, D)),
            full((1, 1)),
        ],
        out_specs=[
            pl.BlockSpec((BLK, D), lambda i: (i, 0)),
            pl.BlockSpec((BLK,), lambda i: (i,)),
        ],
        out_shape=[
            jax.ShapeDtypeStruct((BATCH, D), jnp.float32),
            jax.ShapeDtypeStruct((BATCH,), jnp.float32),
        ],
    )


def kernel(user, item, gmf_user_table, gmf_item_table, mlp_user_table,
           mlp_item_table, mlp_W0, mlp_b0, mlp_W1, mlp_b1,
           hidden_W, hidden_b, nmf_W, nmf_b):
    user = user.astype(jnp.int32)
    item = item.astype(jnp.int32)
    guT = gmf_user_table.T
    giT = gmf_item_table.T
    muT = mlp_user_table.T
    miT = mlp_item_table.T
    tail_u = jnp.concatenate([gmf_user_table[TAIL0:], mlp_user_table[TAIL0:]],
                             axis=1)
    tail_i = jnp.concatenate([gmf_item_table[TAIL0:], mlp_item_table[TAIL0:]],
                             axis=1)
    u_rows, i_rows = _make_sc()(user, item, guT, giT, muT, miT, tail_u, tail_i)
    fused, score = _make_tc()(
        u_rows, i_rows,
        mlp_W0, mlp_b0.reshape(1, -1), mlp_W1, mlp_b1.reshape(1, -1),
        hidden_W, hidden_b.reshape(1, -1),
        nmf_W.reshape(1, -1), nmf_b.reshape(1, 1))
    return (score.reshape(BATCH, 1), fused)


# speedup vs baseline: 2.6897x; 2.6897x over previous
import jax
import jax.numpy as jnp
from jax import lax
from jax.experimental import pallas as pl
from jax.experimental.pallas import tpu as pltpu
from jax.experimental.pallas import tpu_sc as plsc

BATCH = 16384
D = 64
WIDE = 2 * D
NC = 2
NS = 16
NW = NC * NS
NROWS = 100000
CW = 512
NCHUNK = NROWS // CW
TAIL0 = NCHUNK * CW
NTAIL = NROWS - TAIL0
BASE_CH = NCHUNK // NW
EXTRA = NCHUNK - BASE_CH * NW


def _sc_body(user_h, item_h, guT, giT, muT, miT, tail_u, tail_i,
             u_out, i_out,
             sidx, mlist, cbuf_g, cbuf_m, stag, posref, rrbuf,
             sem0, sem1, sem2):
    w = lax.axis_index("s") * NC + lax.axis_index("c")
    lo_chunk = BASE_CH * w + jnp.minimum(w, EXTRA)
    ncz = BASE_CH + (w < EXTRA).astype(jnp.int32)
    lo_col = lo_chunk * CW
    is_last = w == NW - 1
    hi_col = jnp.where(is_last, NROWS, (lo_chunk + ncz) * CW)
    iota = lax.iota(jnp.int32, 16)

    def side(idx_h, tbl_g, tbl_m, tail_t, out):
        pltpu.sync_copy(idx_h, sidx)

        def scan_body(v4, cnt):
            for s in range(4):
                lanes = v4 * 64 + s * 16 + iota
                r = plsc.load_gather(sidx, [lanes])
                m = (r >= lo_col) & (r < hi_col)
                e = (r << 14) | lanes
                plsc.store_compressed(mlist.at[pl.ds(cnt, 16)], e, mask=m)
                cnt = cnt + jnp.sum(m.astype(jnp.int32))
            return cnt

        n = lax.fori_loop(0, BATCH // 64, scan_body, 0)

        def subfilter(c):
            def body(v4, cnt):
                for s in range(4):
                    lanes = v4 * 64 + s * 16 + iota
                    e = plsc.load_gather(mlist, [jnp.minimum(lanes, n - 1)])
                    m = (lanes < n) & ((e >> 23) == c)
                    plsc.store_compressed(sidx.at[pl.ds(cnt, 16)], e, mask=m)
                    cnt = cnt + jnp.sum(m.astype(jnp.int32))
                return cnt

            return lax.fori_loop(0, (n + 63) // 64, body, 0)

        def emit(g_cnt, gather_store):
            def sub_batch(sb, _):
                base = sb * 128
                rrs = []
                for v in range(8):
                    lanes = base + v * 16 + iota
                    e = plsc.load_gather(sidx, [jnp.minimum(lanes, g_cnt - 1)])
                    rrs.append(e >> 14)
                    posref[pl.ds(v * 16, 16)] = e & (BATCH - 1)
                gather_store(rrs)
                pltpu.async_copy(stag, out.at[posref], sem2).wait()
                return 0

            lax.fori_loop(0, (g_cnt + 127) // 128, sub_batch, 0)

        def process_chunk(c_rel, _):
            c = lo_chunk + c_rel
            cps = []
            for k in range(4):
                col = pl.multiple_of((c * 4 + k) * 128, 128)
                dst = pl.ds(64 * k, 64)
                cps.append(pltpu.async_copy(
                    tbl_g.at[:, pl.ds(col, 128)], cbuf_g.at[dst, :], sem0))
                cps.append(pltpu.async_copy(
                    tbl_m.at[:, pl.ds(col, 128)], cbuf_m.at[dst, :], sem1))
            g_cnt = subfilter(c)
            for cp in cps:
                cp.wait()

            def gather_store(rrs):
                lo = lo_chunk * 0 + c * CW

                def dloop(d, _2):
                    dv = jnp.full((16,), d, jnp.int32)
                    for v in range(8):
                        rows = v * 16 + iota
                        rr = rrs[v] - lo
                        di = dv + ((rr >> 7) << 6)
                        cl = rr & 127
                        g = plsc.load_gather(cbuf_g, [di, cl])
                        plsc.store_scatter(stag, [rows, dv], g)
                        m = plsc.load_gather(cbuf_m, [di, cl])
                        plsc.store_scatter(stag, [rows, dv + D], m)
                    return 0

                lax.fori_loop(0, D, dloop, 0)

            emit(g_cnt, gather_store)
            return 0

        lax.fori_loop(0, ncz, process_chunk, 0)

        @pl.when(is_last)
        def _tail():
            g_cnt = subfilter(NCHUNK)

            def sub_batch(sb, _):
                base = sb * 128
                for v in range(8):
                    lanes = base + v * 16 + iota
                    e = plsc.load_gather(sidx, [jnp.minimum(lanes, g_cnt - 1)])
                    rrbuf[pl.ds(v * 16, 16)] = (e >> 14) - TAIL0
                    posref[pl.ds(v * 16, 16)] = e & (BATCH - 1)
                pltpu.async_copy(tail_t.at[rrbuf], stag, sem2).wait()
                pltpu.async_copy(stag, out.at[posref], sem2).wait()
                return 0

            lax.fori_loop(0, (g_cnt + 127) // 128, sub_batch, 0)

    side(user_h, guT, muT, tail_u, u_out)
    side(item_h, giT, miT, tail_i, i_out)


def _make_sc():
    mesh = plsc.VectorSubcoreMesh(core_axis_name="c", subcore_axis_name="s")
    f32, i32 = jnp.float32, jnp.int32
    row = jax.ShapeDtypeStruct((BATCH, WIDE), f32)
    return pl.kernel(
        _sc_body,
        out_type=[row, row],
        mesh=mesh,
        scratch_types=[
            pltpu.VMEM((BATCH,), i32),
            pltpu.VMEM((BATCH,), i32),
            pltpu.VMEM((4 * D, 128), f32),
            pltpu.VMEM((4 * D, 128), f32),
            pltpu.VMEM((128, WIDE), f32),
            pltpu.VMEM((128,), i32),
            pltpu.VMEM((128,), i32),
            pltpu.SemaphoreType.DMA,
            pltpu.SemaphoreType.DMA,
            pltpu.SemaphoreType.DMA,
        ],
        compiler_params=pltpu.CompilerParams(use_tc_tiling_on_sc=True,
                                             needs_layout_passes=False),
    )


BLK = 512


def _tc_body(u, i, w0, b0, w1, b1, hw, hb, nw, nb, fused_o, score_o):
    gu = u[:, :D]
    mu = u[:, D:]
    gi = i[:, :D]
    mi = i[:, D:]
    mlp_x = jnp.concatenate([mu, mi], axis=1)
    h = jnp.maximum(jnp.dot(mlp_x, w0[...],
                            preferred_element_type=jnp.float32) + b0[...], 0.0)
    mlp_out = jnp.maximum(jnp.dot(h, w1[...],
                                  preferred_element_type=jnp.float32) + b1[...], 0.0)
    fused_in = jnp.concatenate([0.5 * gu * gi, 0.5 * mlp_out], axis=1)
    fused = jnp.dot(fused_in, hw[...],
                    preferred_element_type=jnp.float32) + hb[...]
    fused_o[...] = fused
    score_o[...] = jnp.sum(fused * nw[...], axis=1) + nb[0, 0]


def _make_tc():
    grid = (BATCH // BLK,)
    blk_in = pl.BlockSpec((BLK, WIDE), lambda i: (i, 0))
    full = lambda shape: pl.BlockSpec(shape, lambda i: (0, 0))
    return pl.pallas_call(
        _tc_body,
        grid=grid,
        in_specs=[
            blk_in, blk_in,
            full((WIDE, WIDE)),
            full((1, WIDE)),
            full((WIDE, D)),
            full((1, D)),
            full((WIDE, D)),
            full((1, D)),
            full((1, D)),
            full((1, 1)),
        ],
        out_specs=[
            pl.BlockSpec((BLK, D), lambda i: (i, 0)),
            pl.BlockSpec((BLK,), lambda i: (i,)),
        ],
        out_shape=[
            jax.ShapeDtypeStruct((BATCH, D), jnp.float32),
            jax.ShapeDtypeStruct((BATCH,), jnp.float32),
        ],
    )


def kernel(user, item, gmf_user_table, gmf_item_table, mlp_user_table,
           mlp_item_table, mlp_W0, mlp_b0, mlp_W1, mlp_b1,
           hidden_W, hidden_b, nmf_W, nmf_b):
    user = user.astype(jnp.int32)
    item = item.astype(jnp.int32)
    guT = gmf_user_table.T
    giT = gmf_item_table.T
    muT = mlp_user_table.T
    miT = mlp_item_table.T
    tail_u = jnp.concatenate([gmf_user_table[TAIL0:], mlp_user_table[TAIL0:]],
                             axis=1)
    tail_i = jnp.concatenate([gmf_item_table[TAIL0:], mlp_item_table[TAIL0:]],
                             axis=1)
    u_rows, i_rows = _make_sc()(user, item, guT, giT, muT, miT, tail_u, tail_i)
    fused, score = _make_tc()(
        u_rows, i_rows,
        mlp_W0, mlp_b0.reshape(1, -1), mlp_W1, mlp_b1.reshape(1, -1),
        hidden_W, hidden_b.reshape(1, -1),
        nmf_W.reshape(1, -1), nmf_b.reshape(1, 1))
    return (score.reshape(BATCH, 1), fused)
